# baseline (speedup 1.0000x reference)
import functools

import jax
import jax.numpy as jnp
from jax import lax
from jax.experimental import pallas as pl
from jax.experimental.pallas import tpu as pltpu
from jax.experimental.pallas import tpu_sc as plsc

_B = 4
_S = 8192
_D = 768

_NC = 2
_NS = 16
_NW = _NC * _NS

_CHUNK_POS = 16
_CHUNK = _CHUNK_POS * _D
_POS_PER_W = _S // _NW
_CHUNKS_PER_W = _POS_PER_W // _CHUNK_POS
_LANES = 16


def _sc_body(in_hbm, tab_hbm, out_hbm, tab_v, io_v):
    wid = lax.axis_index("s") * _NC + lax.axis_index("c")
    strip_base = wid * _POS_PER_W * _D

    def chunk_body(ci, _):
        tab_off = strip_base + ci * _CHUNK
        pltpu.sync_copy(tab_hbm.at[pl.ds(tab_off, _CHUNK)], tab_v)
        for b in range(_B):
            in_off = b * (_S * _D) + tab_off
            pltpu.sync_copy(in_hbm.at[pl.ds(in_off, _CHUNK)], io_v)

            def add_body(i, _):
                o = i * _LANES
                io_v[pl.ds(o, _LANES)] = (
                    io_v[pl.ds(o, _LANES)] + tab_v[pl.ds(o, _LANES)]
                )
                return 0

            lax.fori_loop(0, _CHUNK // _LANES, add_body, 0)
            pltpu.sync_copy(io_v, out_hbm.at[pl.ds(in_off, _CHUNK)])
        return 0

    lax.fori_loop(0, _CHUNKS_PER_W, chunk_body, 0)


def kernel(inputs, pos_emb_table):
    B, S, D = inputs.shape
    flat_in = inputs.reshape(B * S * D)
    flat_tab = pos_emb_table.reshape(S * D)
    sc_fn = functools.partial(
        pl.kernel,
        mesh=plsc.VectorSubcoreMesh(core_axis_name="c", subcore_axis_name="s"),
        out_type=jax.ShapeDtypeStruct((B * S * D,), jnp.float32),
        scratch_types=[
            pltpu.VMEM((_CHUNK,), jnp.float32),
            pltpu.VMEM((_CHUNK,), jnp.float32),
        ],
    )(_sc_body)
    out = sc_fn(flat_in, flat_tab)
    return out.reshape(B, S, D)


# speedup vs baseline: 1.3837x; 1.3837x over previous
import functools

import jax
import jax.numpy as jnp
from jax import lax
from jax.experimental import pallas as pl
from jax.experimental.pallas import tpu as pltpu
from jax.experimental.pallas import tpu_sc as plsc

_B = 4
_S = 8192
_D = 768

_NC = 2
_NS = 16
_NW = _NC * _NS

_CHUNK_POS = 16
_CHUNK = _CHUNK_POS * _D
_POS_PER_W = _S // _NW
_CHUNKS_PER_W = _POS_PER_W // _CHUNK_POS
_LANES = 16
_UNROLL = 8


def _sc_body(in_hbm, tab_hbm, out_hbm, tab_v, io_v):
    wid = lax.axis_index("s") * _NC + lax.axis_index("c")
    strip_base = wid * _POS_PER_W * _D

    def chunk_body(ci, _):
        tab_off = strip_base + ci * _CHUNK
        pltpu.sync_copy(tab_hbm.at[pl.ds(tab_off, _CHUNK)], tab_v)
        for b in range(_B):
            in_off = b * (_S * _D) + tab_off
            pltpu.sync_copy(in_hbm.at[pl.ds(in_off, _CHUNK)], io_v)

            def add_body(i, _):
                base = i * (_LANES * _UNROLL)
                for j in range(_UNROLL):
                    o = base + j * _LANES
                    io_v[pl.ds(o, _LANES)] = (
                        io_v[pl.ds(o, _LANES)] + tab_v[pl.ds(o, _LANES)]
                    )
                return 0

            lax.fori_loop(0, _CHUNK // (_LANES * _UNROLL), add_body, 0)
            pltpu.sync_copy(io_v, out_hbm.at[pl.ds(in_off, _CHUNK)])
        return 0

    lax.fori_loop(0, _CHUNKS_PER_W, chunk_body, 0)


def kernel(inputs, pos_emb_table):
    B, S, D = inputs.shape
    flat_in = inputs.reshape(B * S * D)
    flat_tab = pos_emb_table.reshape(S * D)
    sc_fn = functools.partial(
        pl.kernel,
        mesh=plsc.VectorSubcoreMesh(core_axis_name="c", subcore_axis_name="s"),
        out_type=jax.ShapeDtypeStruct((B * S * D,), jnp.float32),
        scratch_types=[
            pltpu.VMEM((_CHUNK,), jnp.float32),
            pltpu.VMEM((_CHUNK,), jnp.float32),
        ],
    )(_sc_body)
    out = sc_fn(flat_in, flat_tab)
    return out.reshape(B, S, D)


# speedup vs baseline: 6.8457x; 4.9476x over previous
import jax
import jax.numpy as jnp
from jax.experimental import pallas as pl

_BLOCK_S = 512


def _add_body(x_ref, t_ref, o_ref):
    o_ref[...] = x_ref[...] + t_ref[...]


def kernel(inputs, pos_emb_table):
    B, S, D = inputs.shape
    return pl.pallas_call(
        _add_body,
        grid=(S // _BLOCK_S, B),
        in_specs=[
            pl.BlockSpec((1, _BLOCK_S, D), lambda i, b: (b, i, 0)),
            pl.BlockSpec((_BLOCK_S, D), lambda i, b: (i, 0)),
        ],
        out_specs=pl.BlockSpec((1, _BLOCK_S, D), lambda i, b: (b, i, 0)),
        out_shape=jax.ShapeDtypeStruct((B, S, D), inputs.dtype),
    )(inputs, pos_emb_table)


# speedup vs baseline: 8.4395x; 1.2328x over previous
import jax
import jax.numpy as jnp
from jax.experimental import pallas as pl

_BLOCK_S = 512


def _add_body(x_ref, t_ref, o_ref):
    o_ref[...] = x_ref[...] + t_ref[...][None, :, :]


def kernel(inputs, pos_emb_table):
    B, S, D = inputs.shape
    return pl.pallas_call(
        _add_body,
        grid=(S // _BLOCK_S,),
        in_specs=[
            pl.BlockSpec((B, _BLOCK_S, D), lambda i: (0, i, 0)),
            pl.BlockSpec((_BLOCK_S, D), lambda i: (i, 0)),
        ],
        out_specs=pl.BlockSpec((B, _BLOCK_S, D), lambda i: (0, i, 0)),
        out_shape=jax.ShapeDtypeStruct((B, S, D), inputs.dtype),
    )(inputs, pos_emb_table)
